# baseline (speedup 1.0000x reference)
import jax
import jax.numpy as jnp
from jax import lax
from jax.experimental import pallas as pl
from jax.experimental.pallas import tpu as pltpu
from jax.experimental.pallas import tpu_sc as plsc

NC = 2
NS = 16
NW = NC * NS

B = 128
NCAT = 6
NP = 3
CT = 128
CB = 4
NUNITS = NP * (CT // CB)
U_PER_W = NUNITS // NW
ROW_W = 1024


def _sc_body(emb_hbm, cat_hbm, out_hbm, cat_v, staged, sem_stage, sem_w):
    cid = lax.axis_index("c")
    sid = lax.axis_index("s")
    wid = sid * NC + cid

    pltpu.sync_copy(cat_hbm, cat_v)

    for i in range(U_PER_W):
        u = wid * U_PER_W + i
        np_i = u // (CT // CB)
        cb_i = u % (CT // CB)
        pltpu.async_copy(
            emb_hbm.at[:, np_i, pl.ds(cb_i * CB, CB), :, :],
            staged.at[i], sem_stage)
    for i in range(U_PER_W):
        u = wid * U_PER_W + i
        np_i = u // (CT // CB)
        cb_i = u % (CT // CB)
        pltpu.make_async_copy(
            emb_hbm.at[:, np_i, pl.ds(cb_i * CB, CB), :, :],
            staged.at[i], sem_stage).wait()

    def dst(b, i):
        u = wid * U_PER_W + i
        np_i = u // (CT // CB)
        cb_i = u % (CT // CB)
        return out_hbm.at[b, pl.ds(cb_i * CB, CB), pl.ds(np_i, 1), :]

    def issue(g, carry):
        c16 = cat_v[pl.ds(g * 16, 16)]
        for l in range(16):
            b = g * 16 + l
            c = c16[l]
            for i in range(U_PER_W):
                pltpu.async_copy(staged.at[i, c], dst(b, i), sem_w)
        return carry

    def drain(g, carry):
        c16 = cat_v[pl.ds(g * 16, 16)]
        for l in range(16):
            b = g * 16 + l
            c = c16[l]
            for i in range(U_PER_W):
                pltpu.make_async_copy(staged.at[i, c], dst(b, i), sem_w).wait()
        return carry

    lax.fori_loop(0, B // 16, issue, 0)
    lax.fori_loop(0, B // 16, drain, 0)


def kernel(batch_size, cat_id, embeddings):
    emb4 = embeddings.reshape(NCAT, NP, CT, 1, ROW_W)

    mesh = plsc.VectorSubcoreMesh(core_axis_name="c", subcore_axis_name="s")
    out4 = pl.kernel(
        _sc_body,
        out_type=jax.ShapeDtypeStruct((B, CT, NP, ROW_W), jnp.float32),
        mesh=mesh,
        scratch_types=[
            pltpu.VMEM((B,), jnp.int32),
            pltpu.VMEM((U_PER_W, NCAT, CB, 1, ROW_W), jnp.float32),
            pltpu.SemaphoreType.DMA,
            pltpu.SemaphoreType.DMA,
        ],
    )(emb4, cat_id.astype(jnp.int32))
    return out4.reshape(B, CT, NP * ROW_W)


# speedup vs baseline: 5.5312x; 5.5312x over previous
import jax
import jax.numpy as jnp
from jax import lax
from jax.experimental import pallas as pl
from jax.experimental.pallas import tpu as pltpu
from jax.experimental.pallas import tpu_sc as plsc

NC = 2
NS = 16
NW = NC * NS

B = 128
NCAT = 6
NP = 3
CT = 128
CB = CT // NW
ROW_W = 1024
OUT_W = NP * ROW_W


def _sc_body(emb_hbm, cat_hbm, out_hbm, cat_v, staged, sem_stage, sem_w):
    cid = lax.axis_index("c")
    sid = lax.axis_index("s")
    wid = sid * NC + cid
    ct0 = wid * CB

    pltpu.sync_copy(cat_hbm, cat_v)

    for np_i in range(NP):
        pltpu.async_copy(
            emb_hbm.at[:, np_i, pl.ds(ct0, CB), :],
            staged.at[:, :, pl.ds(np_i * ROW_W, ROW_W)], sem_stage)
    for np_i in range(NP):
        pltpu.make_async_copy(
            emb_hbm.at[:, np_i, pl.ds(ct0, CB), :],
            staged.at[:, :, pl.ds(np_i * ROW_W, ROW_W)], sem_stage).wait()

    def issue(g, carry):
        c16 = cat_v[pl.ds(g * 16, 16)]
        for l in range(16):
            b = g * 16 + l
            c = c16[l]
            pltpu.async_copy(
                staged.at[c], out_hbm.at[b, pl.ds(ct0, CB), :], sem_w)
        return carry

    def drain(g, carry):
        c16 = cat_v[pl.ds(g * 16, 16)]
        for l in range(16):
            b = g * 16 + l
            c = c16[l]
            pltpu.make_async_copy(
                staged.at[c], out_hbm.at[b, pl.ds(ct0, CB), :], sem_w).wait()
        return carry

    lax.fori_loop(0, B // 16, issue, 0)
    lax.fori_loop(0, B // 16, drain, 0)


def kernel(batch_size, cat_id, embeddings):
    emb4 = embeddings.reshape(NCAT, NP, CT, ROW_W)

    mesh = plsc.VectorSubcoreMesh(core_axis_name="c", subcore_axis_name="s")
    out = pl.kernel(
        _sc_body,
        out_type=jax.ShapeDtypeStruct((B, CT, OUT_W), jnp.float32),
        mesh=mesh,
        scratch_types=[
            pltpu.VMEM((B,), jnp.int32),
            pltpu.VMEM((NCAT, CB, OUT_W), jnp.float32),
            pltpu.SemaphoreType.DMA,
            pltpu.SemaphoreType.DMA,
        ],
    )(emb4, cat_id.astype(jnp.int32))
    return out
